# Initial kernel scaffold; baseline (speedup 1.0000x reference)
import jax
import jax.numpy as jnp
from jax.experimental import pallas as pl


def kernel(x, edge_index, edge_attr, batch, params):
    raise NotImplementedError("write your pallas kernel here")


# speedup vs baseline: 2.6183x; 2.6183x over previous
import functools

import jax
import jax.numpy as jnp
import numpy as np
from jax import lax
from jax.experimental import pallas as pl
from jax.experimental.pallas import tpu as pltpu
from jax.experimental.pallas import tpu_sc as plsc

N = 10000
E = 320000
G = 64
NODE_IN = 128
F = 64

NC = 2
NS = 16
NW = NC * NS
EPW = E // NW
K = 80
NCHUNK = EPW // K
NWT = 10
NPT = N // NWT
ZB = 40

_BNS = float(1.0 / np.sqrt(1.0 + 1e-5))


def _sc_body(p_hbm, q_hbm, src_hbm, dst_hbm, st_hbm, z_hbm,
             idx_s, idx_d, qbuf, pbuf, mbuf, zb, stbuf, zshared, sem):
    c = lax.axis_index("c")
    s = lax.axis_index("s")
    w = c * NS + s

    pltpu.sync_copy(st_hbm, stbuf)
    smv = [stbuf[0, pl.ds(16 * j, 16)] for j in range(4)]
    tmv = [stbuf[1, pl.ds(16 * j, 16)] for j in range(4)]

    zero = jnp.zeros((16,), jnp.float32)

    def zrow(r, carry):
        for j in range(8):
            zb[r, pl.ds(16 * j, 16)] = zero
        return carry

    @pl.when(s < NWT)
    def _():
        lax.fori_loop(0, ZB, zrow, 0)

        def zcopy(i, carry):
            pltpu.sync_copy(zb, zshared.at[pl.ds(s * NPT + i * ZB, ZB)])
            return carry

        lax.fori_loop(0, NPT // ZB, zcopy, 0)

    def mrow(e, carry):
        for j in range(4, 8):
            mbuf[e, pl.ds(16 * j, 16)] = zero
        return carry

    lax.fori_loop(0, K, mrow, 0)
    plsc.subcore_barrier()

    def chunk(g, carry):
        base = w * EPW + g * K
        pltpu.sync_copy(src_hbm.at[pl.ds(base, K)], idx_s)
        pltpu.sync_copy(dst_hbm.at[pl.ds(base, K)], idx_d)
        pltpu.sync_copy(q_hbm.at[pl.ds(base, K)], qbuf)
        pltpu.async_copy(p_hbm.at[idx_s], pbuf, sem).wait()

        def ebody(e, carry2):
            for j in range(4):
                sl = pl.ds(16 * j, 16)
                v = qbuf[e, sl] + pbuf[e, sl]
                mbuf[e, sl] = jnp.maximum(v, 0.0) * smv[j] + tmv[j]
            return carry2

        lax.fori_loop(0, K, ebody, 0)
        pltpu.sync_copy(mbuf, zshared.at[idx_d], add=True)
        return carry

    lax.fori_loop(0, NCHUNK, chunk, 0)
    plsc.subcore_barrier()

    @pl.when(s < NWT)
    def _():
        pltpu.sync_copy(zshared.at[pl.ds(s * NPT, NPT)],
                        z_hbm.at[c, pl.ds(s * NPT, NPT)])


_sc_scatter = pl.kernel(
    _sc_body,
    out_type=jax.ShapeDtypeStruct((NC, N, 2 * F), jnp.float32),
    mesh=plsc.VectorSubcoreMesh(core_axis_name="c", subcore_axis_name="s",
                                num_cores=NC, num_subcores=NS),
    scratch_types=[
        pltpu.VMEM((K,), jnp.int32),
        pltpu.VMEM((K,), jnp.int32),
        pltpu.VMEM((K, F), jnp.float32),
        pltpu.VMEM((K, 2 * F), jnp.float32),
        pltpu.VMEM((K, 2 * F), jnp.float32),
        pltpu.VMEM((ZB, 2 * F), jnp.float32),
        pltpu.VMEM((2, F), jnp.float32),
        pltpu.VMEM_SHARED((N, 2 * F), jnp.float32),
        pltpu.SemaphoreType.DMA,
    ],
)


def _embed_body(x_ref, wn_ref, bn_ref, wp_ref, h_ref, p_ref):
    h = jnp.dot(x_ref[...], wn_ref[...],
                preferred_element_type=jnp.float32) + bn_ref[...]
    h_ref[...] = h
    p = jnp.dot(h, wp_ref[...], preferred_element_type=jnp.float32)
    p_ref[...] = jnp.concatenate([p, jnp.zeros_like(p)], axis=1)


def _q_body(attr_ref, we_ref, be_ref, wme_ref, bm_ref, q1_ref, q2_ref,
            q3_ref):
    ea = jnp.dot(attr_ref[...], we_ref[...],
                 preferred_element_type=jnp.float32) + be_ref[...]
    t = jnp.dot(ea, wme_ref[...],
                preferred_element_type=jnp.float32) + bm_ref[...]
    q1_ref[...] = t[:, :F]
    q2_ref[...] = t[:, F:2 * F]
    q3_ref[...] = t[:, 2 * F:]


def _upd_body(h_ref, z_ref, wuh_ref, wua_ref, bu_ref, su_ref, tu_ref,
              wpn_ref, hn_ref, pn_ref):
    aggr = z_ref[0][:, :F] + z_ref[1][:, :F]
    u = (jnp.dot(h_ref[...], wuh_ref[...], preferred_element_type=jnp.float32)
         + jnp.dot(aggr, wua_ref[...], preferred_element_type=jnp.float32)
         + bu_ref[...])
    hn = jnp.maximum(u, 0.0) * su_ref[...] + tu_ref[...]
    hn_ref[...] = hn
    pn = jnp.dot(hn, wpn_ref[...], preferred_element_type=jnp.float32)
    pn_ref[...] = jnp.concatenate([pn, jnp.zeros_like(pn)], axis=1)


def _pool_body(h_ref, b_ref, w1_ref, b1_ref, w2_ref, b2_ref, out_ref):
    nb = N // 8
    g = jnp.zeros((G, F), jnp.float32)
    for rr in range(8):
        brow = b_ref[rr].reshape(1, nb)
        onehot = (lax.broadcasted_iota(jnp.int32, (G, nb), 0) == brow).astype(
            jnp.float32)
        g = g + jnp.dot(onehot, h_ref[pl.ds(rr * nb, nb), :],
                        preferred_element_type=jnp.float32,
                        precision=lax.Precision.HIGHEST)
    r = jnp.maximum(
        jnp.dot(g, w1_ref[...], preferred_element_type=jnp.float32)
        + b1_ref[...], 0.0)
    out_ref[...] = (jnp.dot(r, w2_ref[...],
                            preferred_element_type=jnp.float32)
                    + b2_ref[...])


def _pool_call(h3, batch8, w1, b1, w2, b2):
    return pl.pallas_call(
        _pool_body,
        out_shape=jax.ShapeDtypeStruct((G, 1), jnp.float32),
    )(h3, batch8, w1, b1, w2, b2)


def _final_body(h_ref, z_ref, wuh_ref, wua_ref, bu_ref, su_ref, tu_ref,
                b_ref, w1_ref, b1_ref, w2_ref, b2_ref, out_ref, acc):
    i = pl.program_id(0)

    @pl.when(i == 0)
    def _():
        acc[...] = jnp.zeros_like(acc)

    aggr = z_ref[0][:, :F] + z_ref[1][:, :F]
    u = (jnp.dot(h_ref[...], wuh_ref[...], preferred_element_type=jnp.float32)
         + jnp.dot(aggr, wua_ref[...], preferred_element_type=jnp.float32)
         + bu_ref[...])
    hn = jnp.maximum(u, 0.0) * su_ref[...] + tu_ref[...]
    b = b_ref[0]
    onehot = (lax.broadcasted_iota(jnp.int32, (G, b.shape[1]), 0)
              == b).astype(jnp.float32)
    acc[...] += jnp.dot(onehot, hn, preferred_element_type=jnp.float32)

    @pl.when(i == pl.num_programs(0) - 1)
    def _():
        g = acc[...]
        r = jnp.maximum(
            jnp.dot(g, w1_ref[...], preferred_element_type=jnp.float32)
            + b1_ref[...], 0.0)
        out_ref[...] = (jnp.dot(r, w2_ref[...],
                                preferred_element_type=jnp.float32)
                        + b2_ref[...])


_NB = 10
_BN = N // _NB


def _full(shape):
    return pl.BlockSpec(shape, lambda i: tuple(0 for _ in shape))


def _embed_call(x, wn, bn, wp):
    return pl.pallas_call(
        _embed_body,
        grid=(_NB,),
        in_specs=[
            pl.BlockSpec((_BN, NODE_IN), lambda i: (i, 0)),
            _full((NODE_IN, F)),
            _full((1, F)),
            _full((F, F)),
        ],
        out_specs=[
            pl.BlockSpec((_BN, F), lambda i: (i, 0)),
            pl.BlockSpec((_BN, 2 * F), lambda i: (i, 0)),
        ],
        out_shape=[
            jax.ShapeDtypeStruct((N, F), jnp.float32),
            jax.ShapeDtypeStruct((N, 2 * F), jnp.float32),
        ],
    )(x, wn, bn, wp)


_EB = 4000
_NEB = E // _EB


def _q_call(attr, we, be, wme, bm):
    return pl.pallas_call(
        _q_body,
        grid=(_NEB,),
        in_specs=[
            pl.BlockSpec((_EB, 16), lambda i: (i, 0)),
            _full((16, 32)),
            _full((1, 32)),
            _full((32, 3 * F)),
            _full((1, 3 * F)),
        ],
        out_specs=[pl.BlockSpec((_EB, F), lambda i: (i, 0))] * 3,
        out_shape=[jax.ShapeDtypeStruct((E, F), jnp.float32)] * 3,
    )(attr, we, be, wme, bm)


def _upd_call(h, z, wuh, wua, bu, su, tu, wpn):
    return pl.pallas_call(
        _upd_body,
        grid=(_NB,),
        in_specs=[
            pl.BlockSpec((_BN, F), lambda i: (i, 0)),
            pl.BlockSpec((NC, _BN, 2 * F), lambda i: (0, i, 0)),
            _full((F, F)),
            _full((F, F)),
            _full((1, F)),
            _full((1, F)),
            _full((1, F)),
            _full((F, F)),
        ],
        out_specs=[
            pl.BlockSpec((_BN, F), lambda i: (i, 0)),
            pl.BlockSpec((_BN, 2 * F), lambda i: (i, 0)),
        ],
        out_shape=[
            jax.ShapeDtypeStruct((N, F), jnp.float32),
            jax.ShapeDtypeStruct((N, 2 * F), jnp.float32),
        ],
    )(h, z, wuh, wua, bu, su, tu, wpn)


def _final_call(h, z, wuh, wua, bu, su, tu, batch3, w1, b1, w2, b2):
    return pl.pallas_call(
        _final_body,
        grid=(_NB,),
        in_specs=[
            pl.BlockSpec((_BN, F), lambda i: (i, 0)),
            pl.BlockSpec((NC, _BN, 2 * F), lambda i: (0, i, 0)),
            _full((F, F)),
            _full((F, F)),
            _full((1, F)),
            _full((1, F)),
            _full((1, F)),
            pl.BlockSpec((1, 1, _BN), lambda i: (i, 0, 0)),
            _full((F, F)),
            _full((1, F)),
            _full((F, 1)),
            _full((1, 1)),
        ],
        out_specs=pl.BlockSpec((G, 1), lambda i: (0, 0)),
        out_shape=jax.ShapeDtypeStruct((G, 1), jnp.float32),
        scratch_shapes=[pltpu.VMEM((G, F), jnp.float32)],
    )(h, z, wuh, wua, bu, su, tu, batch3, w1, b1, w2, b2)


def kernel(x, edge_index, edge_attr, batch, params):
    wn, bn = params["node_emb"]
    we, be = params["edge_emb"]
    layers = params["layers"]

    wme = jnp.concatenate([lp["Wm"][F:] for lp in layers], axis=1)
    bmc = jnp.concatenate([lp["bm"] for lp in layers]).reshape(1, 3 * F)

    src = edge_index[0]
    dst = edge_index[1]

    h, p = _embed_call(x, wn, bn.reshape(1, F), layers[0]["Wm"][:F])
    qs = _q_call(edge_attr, we, be.reshape(1, 32), wme, bmc)

    for li, lp in enumerate(layers):
        st = jnp.stack([lp["gm"] * _BNS, lp["betam"]])
        z = _sc_scatter(p, qs[li], src, dst, st)
        wu = lp["Wu"]
        bu = lp["bu"].reshape(1, F)
        su = (lp["gu"] * _BNS).reshape(1, F)
        tu = lp["betau"].reshape(1, F)
        if li < 2:
            h, p = _upd_call(h, z, wu[:F], wu[F:], bu, su, tu,
                             layers[li + 1]["Wm"][:F])
        else:
            h3, _ = _upd_call(h, z, wu[:F], wu[F:], bu, su, tu,
                              layers[0]["Wm"][:F])
            out = _pool_call(
                h3, batch.reshape(8, N // 8),
                params["r1"][0], params["r1"][1].reshape(1, F),
                params["r2"][0], params["r2"][1].reshape(1, 1))
    return out


# speedup vs baseline: 2.6252x; 1.0026x over previous
import jax
import jax.numpy as jnp
import numpy as np
from jax import lax
from jax.experimental import pallas as pl
from jax.experimental.pallas import tpu as pltpu
from jax.experimental.pallas import tpu_sc as plsc

N = 10000
E = 320000
G = 64
NODE_IN = 128
F = 64

NC = 2
NS = 16
NW = NC * NS
EPW = E // NW
K = 80
NCHUNK = EPW // K
NWT = 10
NPT = N // NWT
ZB = 40

_BNS = float(1.0 / np.sqrt(1.0 + 1e-5))


def _sc_body(p_hbm, q_hbm, src_hbm, dst_hbm, st_hbm, z_hbm,
             idx_s, idx_d, qbuf, pbuf, mbuf, zb, stbuf, zshared, sem):
    c = lax.axis_index("c")
    s = lax.axis_index("s")
    w = c * NS + s

    pltpu.sync_copy(st_hbm, stbuf)
    smv = [stbuf[0, pl.ds(16 * j, 16)] for j in range(4)]
    tmv = [stbuf[1, pl.ds(16 * j, 16)] for j in range(4)]

    zero = jnp.zeros((16,), jnp.float32)

    def zrow(r, carry):
        for j in range(8):
            zb[r, pl.ds(16 * j, 16)] = zero
        return carry

    @pl.when(s < NWT)
    def _():
        lax.fori_loop(0, ZB, zrow, 0)

        def zcopy(i, carry):
            pltpu.sync_copy(zb, zshared.at[pl.ds(s * NPT + i * ZB, ZB)])
            return carry

        lax.fori_loop(0, NPT // ZB, zcopy, 0)

    def mrow(e, carry):
        for j in range(4, 8):
            mbuf[e, pl.ds(16 * j, 16)] = zero
        return carry

    lax.fori_loop(0, K, mrow, 0)
    plsc.subcore_barrier()

    def chunk(g, carry):
        base = w * EPW + g * K
        pltpu.sync_copy(src_hbm.at[pl.ds(base, K)], idx_s)
        pltpu.sync_copy(dst_hbm.at[pl.ds(base, K)], idx_d)
        pltpu.sync_copy(q_hbm.at[pl.ds(base, K)], qbuf)
        pltpu.async_copy(p_hbm.at[idx_s], pbuf, sem).wait()

        def ebody(e, carry2):
            for j in range(4):
                sl = pl.ds(16 * j, 16)
                v = qbuf[e, sl] + pbuf[e, sl]
                mbuf[e, sl] = jnp.maximum(v, 0.0) * smv[j] + tmv[j]
            return carry2

        lax.fori_loop(0, K, ebody, 0)
        pltpu.sync_copy(mbuf, zshared.at[idx_d], add=True)
        return carry

    lax.fori_loop(0, NCHUNK, chunk, 0)
    plsc.subcore_barrier()

    @pl.when(s < NWT)
    def _():
        pltpu.sync_copy(zshared.at[pl.ds(s * NPT, NPT)],
                        z_hbm.at[c, pl.ds(s * NPT, NPT)])


_sc_scatter = pl.kernel(
    _sc_body,
    out_type=jax.ShapeDtypeStruct((NC, N, 2 * F), jnp.float32),
    mesh=plsc.VectorSubcoreMesh(core_axis_name="c", subcore_axis_name="s",
                                num_cores=NC, num_subcores=NS),
    scratch_types=[
        pltpu.VMEM((K,), jnp.int32),
        pltpu.VMEM((K,), jnp.int32),
        pltpu.VMEM((K, F), jnp.float32),
        pltpu.VMEM((K, 2 * F), jnp.float32),
        pltpu.VMEM((K, 2 * F), jnp.float32),
        pltpu.VMEM((ZB, 2 * F), jnp.float32),
        pltpu.VMEM((2, F), jnp.float32),
        pltpu.VMEM_SHARED((N, 2 * F), jnp.float32),
        pltpu.SemaphoreType.DMA,
    ],
)


def _embed_body(x_ref, wn_ref, bn_ref, wp_ref, h_ref, p_ref):
    h = jnp.dot(x_ref[...], wn_ref[...],
                preferred_element_type=jnp.float32) + bn_ref[...]
    h_ref[...] = h
    p = jnp.dot(h, wp_ref[...], preferred_element_type=jnp.float32)
    p_ref[...] = jnp.concatenate([p, jnp.zeros_like(p)], axis=1)


def _q_body(attr_ref, we_ref, be_ref, wme_ref, bm_ref, q1_ref, q2_ref,
            q3_ref):
    ea = jnp.dot(attr_ref[...], we_ref[...],
                 preferred_element_type=jnp.float32) + be_ref[...]
    t = jnp.dot(ea, wme_ref[...],
                preferred_element_type=jnp.float32) + bm_ref[...]
    q1_ref[...] = t[:, :F]
    q2_ref[...] = t[:, F:2 * F]
    q3_ref[...] = t[:, 2 * F:]


def _upd_body(h_ref, z_ref, wuh_ref, wua_ref, bu_ref, su_ref, tu_ref,
              wpn_ref, hn_ref, pn_ref):
    aggr = z_ref[0][:, :F] + z_ref[1][:, :F]
    u = (jnp.dot(h_ref[...], wuh_ref[...], preferred_element_type=jnp.float32)
         + jnp.dot(aggr, wua_ref[...], preferred_element_type=jnp.float32)
         + bu_ref[...])
    hn = jnp.maximum(u, 0.0) * su_ref[...] + tu_ref[...]
    hn_ref[...] = hn
    pn = jnp.dot(hn, wpn_ref[...], preferred_element_type=jnp.float32)
    pn_ref[...] = jnp.concatenate([pn, jnp.zeros_like(pn)], axis=1)


def _pool_body(h_ref, b_ref, w1_ref, b1_ref, w2_ref, b2_ref, out_ref):
    nb = N // 8
    g = jnp.zeros((G, F), jnp.float32)
    for rr in range(8):
        brow = b_ref[rr].reshape(1, nb)
        onehot = (lax.broadcasted_iota(jnp.int32, (G, nb), 0) == brow).astype(
            jnp.float32)
        g = g + jnp.dot(onehot, h_ref[pl.ds(rr * nb, nb), :],
                        preferred_element_type=jnp.float32,
                        precision=lax.Precision.HIGHEST)
    r = jnp.maximum(
        jnp.dot(g, w1_ref[...], preferred_element_type=jnp.float32)
        + b1_ref[...], 0.0)
    out_ref[...] = (jnp.dot(r, w2_ref[...],
                            preferred_element_type=jnp.float32)
                    + b2_ref[...])


def _pool_call(h3, batch8, w1, b1, w2, b2):
    return pl.pallas_call(
        _pool_body,
        out_shape=jax.ShapeDtypeStruct((G, 1), jnp.float32),
    )(h3, batch8, w1, b1, w2, b2)


_NB = 10
_BN = N // _NB


def _full(shape):
    return pl.BlockSpec(shape, lambda i: tuple(0 for _ in shape))


def _embed_call(x, wn, bn, wp):
    return pl.pallas_call(
        _embed_body,
        grid=(_NB,),
        in_specs=[
            pl.BlockSpec((_BN, NODE_IN), lambda i: (i, 0)),
            _full((NODE_IN, F)),
            _full((1, F)),
            _full((F, F)),
        ],
        out_specs=[
            pl.BlockSpec((_BN, F), lambda i: (i, 0)),
            pl.BlockSpec((_BN, 2 * F), lambda i: (i, 0)),
        ],
        out_shape=[
            jax.ShapeDtypeStruct((N, F), jnp.float32),
            jax.ShapeDtypeStruct((N, 2 * F), jnp.float32),
        ],
    )(x, wn, bn, wp)


_EB = 4000
_NEB = E // _EB


def _q_call(attr, we, be, wme, bm):
    return pl.pallas_call(
        _q_body,
        grid=(_NEB,),
        in_specs=[
            pl.BlockSpec((_EB, 16), lambda i: (i, 0)),
            _full((16, 32)),
            _full((1, 32)),
            _full((32, 3 * F)),
            _full((1, 3 * F)),
        ],
        out_specs=[pl.BlockSpec((_EB, F), lambda i: (i, 0))] * 3,
        out_shape=[jax.ShapeDtypeStruct((E, F), jnp.float32)] * 3,
    )(attr, we, be, wme, bm)


def _upd_call(h, z, wuh, wua, bu, su, tu, wpn):
    return pl.pallas_call(
        _upd_body,
        grid=(_NB,),
        in_specs=[
            pl.BlockSpec((_BN, F), lambda i: (i, 0)),
            pl.BlockSpec((NC, _BN, 2 * F), lambda i: (0, i, 0)),
            _full((F, F)),
            _full((F, F)),
            _full((1, F)),
            _full((1, F)),
            _full((1, F)),
            _full((F, F)),
        ],
        out_specs=[
            pl.BlockSpec((_BN, F), lambda i: (i, 0)),
            pl.BlockSpec((_BN, 2 * F), lambda i: (i, 0)),
        ],
        out_shape=[
            jax.ShapeDtypeStruct((N, F), jnp.float32),
            jax.ShapeDtypeStruct((N, 2 * F), jnp.float32),
        ],
    )(h, z, wuh, wua, bu, su, tu, wpn)


def kernel(x, edge_index, edge_attr, batch, params):
    wn, bn = params["node_emb"]
    we, be = params["edge_emb"]
    layers = params["layers"]

    wme = jnp.concatenate([lp["Wm"][F:] for lp in layers], axis=1)
    bmc = jnp.concatenate([lp["bm"] for lp in layers]).reshape(1, 3 * F)

    src = edge_index[0]
    dst = edge_index[1]

    h, p = _embed_call(x, wn, bn.reshape(1, F), layers[0]["Wm"][:F])
    qs = _q_call(edge_attr, we, be.reshape(1, 32), wme, bmc)

    for li, lp in enumerate(layers):
        st = jnp.stack([lp["gm"] * _BNS, lp["betam"]])
        z = _sc_scatter(p, qs[li], src, dst, st)
        wu = lp["Wu"]
        bu = lp["bu"].reshape(1, F)
        su = (lp["gu"] * _BNS).reshape(1, F)
        tu = lp["betau"].reshape(1, F)
        if li < 2:
            h, p = _upd_call(h, z, wu[:F], wu[F:], bu, su, tu,
                             layers[li + 1]["Wm"][:F])
        else:
            h3, _ = _upd_call(h, z, wu[:F], wu[F:], bu, su, tu,
                              layers[0]["Wm"][:F])
            out = _pool_call(
                h3, batch.reshape(8, N // 8),
                params["r1"][0], params["r1"][1].reshape(1, F),
                params["r2"][0], params["r2"][1].reshape(1, 1))
    return out
